# baseline (speedup 1.0000x reference)
import functools

import jax
import jax.numpy as jnp
from jax import lax
from jax.experimental import pallas as pl
from jax.experimental.pallas import tpu as pltpu
from jax.experimental.pallas import tpu_sc as plsc

_DELTA_PULL = 0.5
_DELTA_PUSH = 1.5
_W_PULL = 1.0
_W_PUSH = 1.0
_W_REG = 0.001
_EPS = 1e-12
_K = 48
_KP = 48
_CG = 512


def _sc_body(emb_ref, lab_ref, out_ref, lbuf, xbuf, acc, *, b, e, n, nw):
    p = n // nw
    wid = lax.axis_index("s") * 2 + lax.axis_index("c")
    base = wid * p
    onesf = jnp.full((16,), 1.0, jnp.float32)
    zf = jnp.zeros((16,), jnp.float32)

    for bi in range(b):
        def zero_row(r, _):
            acc[pl.ds(r * 16, 16)] = zf
            return 0
        lax.fori_loop(0, (_K * _KP) // 16, zero_row, 0)

        pltpu.sync_copy(lab_ref.at[bi, pl.ds(base, p)], lbuf)

        def chunk(c, _):
            lo = base + c * _CG
            pltpu.sync_copy(emb_ref.at[bi, :, pl.ds(lo, _CG)], xbuf)

            def group(g, _):
                lblv = lbuf[pl.ds(c * _CG + g * 16, 16)]
                idx0 = lblv * _KP
                for j in range(e):
                    xv = xbuf[j, pl.ds(g * 16, 16)]
                    plsc.addupdate_scatter(acc, [idx0 + j], xv)
                plsc.addupdate_scatter(acc, [idx0 + e], onesf)
                return 0

            lax.fori_loop(0, _CG // 16, group, 0)
            return 0

        lax.fori_loop(0, p // _CG, chunk, 0)
        pltpu.sync_copy(acc, out_ref.at[pl.ds((bi * nw + wid) * _K * _KP,
                                              _K * _KP)])


def _tc_body(emb_ref, lab_ref, part_ref, lp_ref, lq_ref, lr_ref, kp_ref,
             sums_s, counts_s, cl_s, *, nt, nw):
    t = pl.program_id(1)
    tt = emb_ref.shape[2]
    e = emb_ref.shape[1]
    cc = 512 if tt % 512 == 0 else tt
    nck = tt // cc
    kiota = lax.broadcasted_iota(jnp.int32, (_K, cc), 0)

    @pl.when(t == 0)
    def _init():
        pr = part_ref[...]
        psum = jnp.sum(pr.reshape(nw, _K, _KP), axis=0)
        sums_s[...] = psum[:, 0:e]
        counts_s[...] = psum[:, e:e + 1]
        cl_s[...] = jnp.zeros_like(cl_s)

    counts_c = jnp.maximum(counts_s[...], 1.0)
    centers = sums_s[...] / counts_c
    cn2 = jnp.sum(centers * centers, axis=1, keepdims=True)
    for c in range(nck):
        x = emb_ref[0, :, c * cc:(c + 1) * cc]
        lbl = lab_ref[0, 0, c * cc:(c + 1) * cc]
        oh = (lbl[None, :] == kiota).astype(jnp.float32)
        dots = lax.dot_general(
            centers, x, (((1,), (0,)), ((), ())),
            preferred_element_type=jnp.float32)
        sel = jnp.sum(oh * (dots - 0.5 * cn2), axis=0)
        en2 = jnp.sum(x * x, axis=0)
        d2 = jnp.maximum(en2 - 2.0 * sel, 0.0) + _EPS
        dist = jnp.sqrt(d2)
        hinged = jnp.where(lbl > 0,
                           jnp.maximum(dist - _DELTA_PULL, 0.0) ** 2,
                           0.0)
        cl_s[...] += jnp.sum(oh * hinged[None, :], axis=1, keepdims=True)

    @pl.when(t == nt - 1)
    def _finalize():
        counts_raw = counts_s[...]
        counts_cc = jnp.maximum(counts_raw, 1.0)
        cen = sums_s[...] / counts_cc
        kidx = lax.broadcasted_iota(jnp.int32, (_K, 1), 0)
        pf = jnp.where((counts_raw > 0.0) & (kidx >= 1), 1.0, 0.0)
        kp = jnp.sum(pf)
        kf = jnp.maximum(kp, 1.0)
        cen2 = jnp.sum(cen * cen, axis=1, keepdims=True)
        l_pull = jnp.sum(pf * (cl_s[...] / counts_cc)) / kf
        norms = jnp.sqrt(cen2 + _EPS)
        l_reg = jnp.sum(pf * norms) / kf
        gram = lax.dot_general(
            cen, cen, (((1,), (1,)), ((), ())),
            preferred_element_type=jnp.float32)
        cn2_row = lax.dot_general(
            jnp.ones((1, cen.shape[1]), jnp.float32), cen * cen,
            (((1,), (1,)), ((), ())),
            preferred_element_type=jnp.float32)
        pw2 = jnp.maximum(cen2 + cn2_row - 2.0 * gram, 0.0)
        pw = jnp.sqrt(pw2 + _EPS)
        ii = lax.broadcasted_iota(jnp.int32, (_K, _K), 0)
        jj = lax.broadcasted_iota(jnp.int32, (_K, _K), 1)
        pair_f = lax.dot_general(
            pf, pf, (((1,), (1,)), ((), ())),
            preferred_element_type=jnp.float32)
        pair_f = pair_f * jnp.where(jj > ii, 1.0, 0.0)
        hv = pair_f * jnp.maximum(2.0 * _DELTA_PUSH - pw, 0.0) ** 2
        npairs = jnp.sum(pair_f)
        l_push = jnp.where(npairs > 0.0,
                           jnp.sum(hv) / jnp.maximum(npairs, 1.0),
                           0.0)
        lp_ref[...] = jnp.reshape(l_pull, (1, 1, 1))
        lq_ref[...] = jnp.reshape(l_push, (1, 1, 1))
        lr_ref[...] = jnp.reshape(l_reg, (1, 1, 1))
        kp_ref[...] = jnp.reshape(kp, (1, 1, 1))


def kernel(embedding, ins_label):
    b, e = embedding.shape[0], embedding.shape[1]
    n = embedding.shape[2] * embedding.shape[3]
    t = 8192 if n % 8192 == 0 else n
    nt = n // t
    emb = embedding.reshape(b, e, n)
    lab2 = ins_label.reshape(b, n)
    lab = ins_label.reshape(b * nt, 1, t)

    info = plsc.get_sparse_core_info()
    nw = info.num_cores * info.num_subcores
    p = n // nw

    sc_seg = functools.partial(
        pl.kernel,
        mesh=plsc.VectorSubcoreMesh(core_axis_name="c", subcore_axis_name="s"),
        out_type=jax.ShapeDtypeStruct((b * nw * _K * _KP,), jnp.float32),
        compiler_params=pltpu.CompilerParams(needs_layout_passes=False),
        scratch_types=[
            pltpu.VMEM((p,), jnp.int32),
            pltpu.VMEM((e, _CG), jnp.float32),
            pltpu.VMEM((_K * _KP,), jnp.float32),
        ],
    )(functools.partial(_sc_body, b=b, e=e, n=n, nw=nw))
    partials = sc_seg(emb, lab2).reshape(b * nw * _K, _KP)

    out_shape = [jax.ShapeDtypeStruct((b, 1, 1), jnp.float32)] * 4
    out_spec = pl.BlockSpec((1, 1, 1), lambda bi, ti: (bi, 0, 0))
    lp, lq, lr, kp = pl.pallas_call(
        functools.partial(_tc_body, nt=nt, nw=nw),
        grid=(b, nt),
        in_specs=[
            pl.BlockSpec((1, e, t), lambda bi, ti: (bi, 0, ti)),
            pl.BlockSpec((1, 1, t), lambda bi, ti: (bi * nt + ti, 0, 0)),
            pl.BlockSpec((nw * _K, _KP), lambda bi, ti: (bi, 0)),
        ],
        out_specs=[out_spec] * 4,
        out_shape=out_shape,
        scratch_shapes=[
            pltpu.VMEM((_K, e), jnp.float32),
            pltpu.VMEM((_K, 1), jnp.float32),
            pltpu.VMEM((_K, 1), jnp.float32),
        ],
    )(emb, lab, partials)

    lp = lp.reshape(b)
    lq = lq.reshape(b)
    lr = lr.reshape(b)
    kp = kp.reshape(b)
    has = (kp > 0.0).astype(jnp.float32)
    nvalid = jnp.maximum(jnp.sum(has), 1.0)
    l_pull = jnp.sum(has * lp) / nvalid
    l_push = jnp.sum(has * lq) / nvalid
    l_reg = jnp.sum(has * lr) / nvalid
    total = _W_PULL * l_pull + _W_PUSH * l_push + _W_REG * l_reg
    return {"loss": total, "l_pull": l_pull, "l_push": l_push,
            "l_reg": l_reg}


# speedup vs baseline: 1.3134x; 1.3134x over previous
import functools

import jax
import jax.numpy as jnp
from jax import lax
from jax.experimental import pallas as pl
from jax.experimental.pallas import tpu as pltpu
from jax.experimental.pallas import tpu_sc as plsc

_DELTA_PULL = 0.5
_DELTA_PUSH = 1.5
_W_PULL = 1.0
_W_PUSH = 1.0
_W_REG = 0.001
_EPS = 1e-12
_K = 48
_CG = 512


def _sc_body(emb_ref, lab_ref, out_ref, lbuf, xbuf, *accs, b, e, n, nw):
    p = n // nw
    wid = lax.axis_index("s") * 2 + lax.axis_index("c")
    base = wid * p
    nb = e + 1
    onesf = jnp.full((16,), 1.0, jnp.float32)
    zf = jnp.zeros((16,), jnp.float32)

    for bi in range(b):
        for j in range(nb):
            for c0 in range(0, _K, 16):
                accs[j][pl.ds(c0, 16)] = zf

        pltpu.sync_copy(lab_ref.at[bi, pl.ds(base, p)], lbuf)

        def chunk(c, _):
            lo = base + c * _CG
            pltpu.sync_copy(emb_ref.at[bi, :, pl.ds(lo, _CG)], xbuf)

            def group(g, _):
                lblv = lbuf[pl.ds(c * _CG + g * 16, 16)]
                for j in range(e):
                    xv = xbuf[j, pl.ds(g * 16, 16)]
                    plsc.addupdate_scatter(accs[j], [lblv], xv)
                plsc.addupdate_scatter(accs[e], [lblv], onesf)
                return 0

            lax.fori_loop(0, _CG // 16, group, 0)
            return 0

        lax.fori_loop(0, p // _CG, chunk, 0)
        for j in range(nb):
            pltpu.sync_copy(
                accs[j],
                out_ref.at[pl.ds(((bi * nw + wid) * nb + j) * _K, _K)])


def _tc_body(emb_ref, lab_ref, part_ref, lp_ref, lq_ref, lr_ref, kp_ref,
             sumsT_s, counts_s, cl_s, *, nt, nw):
    t = pl.program_id(1)
    tt = emb_ref.shape[2]
    e = emb_ref.shape[1]
    nb = e + 1
    cc = 512 if tt % 512 == 0 else tt
    nck = tt // cc
    kiota = lax.broadcasted_iota(jnp.int32, (_K, cc), 0)
    unit_cnt = (lax.broadcasted_iota(jnp.int32, (nb, 1), 0) == e
                ).astype(jnp.float32)

    @pl.when(t == 0)
    def _init():
        pr = part_ref[...]
        psum = jnp.sum(pr.reshape(nw, nb, _K), axis=0)
        sumsT_s[...] = psum[0:e, :]
        counts_s[...] = lax.dot_general(
            psum, unit_cnt, (((0,), (0,)), ((), ())),
            preferred_element_type=jnp.float32)
        cl_s[...] = jnp.zeros_like(cl_s)

    counts_col = jnp.maximum(counts_s[...], 1.0)
    counts_row = counts_col.reshape(1, _K)
    centersT = sumsT_s[...] / counts_row
    cn2_col = lax.dot_general(
        centersT * centersT, jnp.ones((e, 1), jnp.float32),
        (((0,), (0,)), ((), ())),
        preferred_element_type=jnp.float32)
    for c in range(nck):
        x = emb_ref[0, :, c * cc:(c + 1) * cc]
        lbl = lab_ref[0, 0, c * cc:(c + 1) * cc]
        oh = (lbl[None, :] == kiota).astype(jnp.float32)
        dots = lax.dot_general(
            centersT, x, (((0,), (0,)), ((), ())),
            preferred_element_type=jnp.float32)
        sel = jnp.sum(oh * (dots - 0.5 * cn2_col), axis=0)
        en2 = jnp.sum(x * x, axis=0)
        d2 = jnp.maximum(en2 - 2.0 * sel, 0.0) + _EPS
        dist = jnp.sqrt(d2)
        hinged = jnp.where(lbl > 0,
                           jnp.maximum(dist - _DELTA_PULL, 0.0) ** 2,
                           0.0)
        cl_s[...] += jnp.sum(oh * hinged[None, :], axis=1, keepdims=True)

    @pl.when(t == nt - 1)
    def _finalize():
        counts_cc = jnp.maximum(counts_s[...], 1.0)
        cenT = sumsT_s[...] / counts_cc.reshape(1, _K)
        kidx = lax.broadcasted_iota(jnp.int32, (_K, 1), 0)
        pf = jnp.where((counts_s[...] > 0.0) & (kidx >= 1), 1.0, 0.0)
        kp = jnp.sum(pf)
        kf = jnp.maximum(kp, 1.0)
        cen2 = lax.dot_general(
            cenT * cenT, jnp.ones((e, 1), jnp.float32),
            (((0,), (0,)), ((), ())),
            preferred_element_type=jnp.float32)
        l_pull = jnp.sum(pf * (cl_s[...] / counts_cc)) / kf
        norms = jnp.sqrt(cen2 + _EPS)
        l_reg = jnp.sum(pf * norms) / kf
        gram = lax.dot_general(
            cenT, cenT, (((0,), (0,)), ((), ())),
            preferred_element_type=jnp.float32)
        pw2 = jnp.maximum(cen2 + cen2.reshape(1, _K) - 2.0 * gram, 0.0)
        pw = jnp.sqrt(pw2 + _EPS)
        ii = lax.broadcasted_iota(jnp.int32, (_K, _K), 0)
        jj = lax.broadcasted_iota(jnp.int32, (_K, _K), 1)
        pair_f = lax.dot_general(
            pf, pf, (((1,), (1,)), ((), ())),
            preferred_element_type=jnp.float32)
        pair_f = pair_f * jnp.where(jj > ii, 1.0, 0.0)
        hv = pair_f * jnp.maximum(2.0 * _DELTA_PUSH - pw, 0.0) ** 2
        npairs = jnp.sum(pair_f)
        l_push = jnp.where(npairs > 0.0,
                           jnp.sum(hv) / jnp.maximum(npairs, 1.0),
                           0.0)
        lp_ref[...] = jnp.reshape(l_pull, (1, 1, 1))
        lq_ref[...] = jnp.reshape(l_push, (1, 1, 1))
        lr_ref[...] = jnp.reshape(l_reg, (1, 1, 1))
        kp_ref[...] = jnp.reshape(kp, (1, 1, 1))


def kernel(embedding, ins_label):
    b, e = embedding.shape[0], embedding.shape[1]
    n = embedding.shape[2] * embedding.shape[3]
    t = 8192 if n % 8192 == 0 else n
    nt = n // t
    nb = e + 1
    emb = embedding.reshape(b, e, n)
    lab2 = ins_label.reshape(b, n)
    lab = ins_label.reshape(b * nt, 1, t)

    info = plsc.get_sparse_core_info()
    nw = info.num_cores * info.num_subcores
    p = n // nw

    sc_seg = functools.partial(
        pl.kernel,
        mesh=plsc.VectorSubcoreMesh(core_axis_name="c", subcore_axis_name="s"),
        out_type=jax.ShapeDtypeStruct((b * nw * nb * _K,), jnp.float32),
        compiler_params=pltpu.CompilerParams(needs_layout_passes=False),
        scratch_types=[
            pltpu.VMEM((p,), jnp.int32),
            pltpu.VMEM((e, _CG), jnp.float32),
        ] + [pltpu.VMEM((_K,), jnp.float32) for _ in range(nb)],
    )(functools.partial(_sc_body, b=b, e=e, n=n, nw=nw))
    partials = sc_seg(emb, lab2).reshape(b * nw * nb, _K)

    out_shape = [jax.ShapeDtypeStruct((b, 1, 1), jnp.float32)] * 4
    out_spec = pl.BlockSpec((1, 1, 1), lambda bi, ti: (bi, 0, 0))
    lp, lq, lr, kp = pl.pallas_call(
        functools.partial(_tc_body, nt=nt, nw=nw),
        grid=(b, nt),
        in_specs=[
            pl.BlockSpec((1, e, t), lambda bi, ti: (bi, 0, ti)),
            pl.BlockSpec((1, 1, t), lambda bi, ti: (bi * nt + ti, 0, 0)),
            pl.BlockSpec((nw * nb, _K), lambda bi, ti: (bi, 0)),
        ],
        out_specs=[out_spec] * 4,
        out_shape=out_shape,
        scratch_shapes=[
            pltpu.VMEM((e, _K), jnp.float32),
            pltpu.VMEM((_K, 1), jnp.float32),
            pltpu.VMEM((_K, 1), jnp.float32),
        ],
    )(emb, lab, partials)

    lp = lp.reshape(b)
    lq = lq.reshape(b)
    lr = lr.reshape(b)
    kp = kp.reshape(b)
    has = (kp > 0.0).astype(jnp.float32)
    nvalid = jnp.maximum(jnp.sum(has), 1.0)
    l_pull = jnp.sum(has * lp) / nvalid
    l_push = jnp.sum(has * lq) / nvalid
    l_reg = jnp.sum(has * lr) / nvalid
    total = _W_PULL * l_pull + _W_PUSH * l_push + _W_REG * l_reg
    return {"loss": total, "l_pull": l_pull, "l_push": l_push,
            "l_reg": l_reg}


# speedup vs baseline: 1.6013x; 1.2193x over previous
import functools

import jax
import jax.numpy as jnp
from jax import lax
from jax.experimental import pallas as pl
from jax.experimental.pallas import tpu as pltpu
from jax.experimental.pallas import tpu_sc as plsc

_DELTA_PULL = 0.5
_DELTA_PUSH = 1.5
_W_PULL = 1.0
_W_PUSH = 1.0
_W_REG = 0.001
_EPS = 1e-12
_K = 48
_CG = 512


def _sc_body(emb_ref, lab_ref, out_ref, lbuf, xbuf, *accs, b, e, n, nw):
    p = n // nw
    wid = lax.axis_index("s") * 2 + lax.axis_index("c")
    base = wid * p
    nb = e + 1
    onesf = jnp.full((16,), 1.0, jnp.float32)
    zf = jnp.zeros((16,), jnp.float32)

    for bi in range(b):
        for j in range(nb):
            for c0 in range(0, _K, 16):
                accs[j][pl.ds(c0, 16)] = zf

        pltpu.sync_copy(lab_ref.at[bi, pl.ds(base, p)], lbuf)

        def chunk(c, _):
            lo = base + c * _CG
            pltpu.sync_copy(emb_ref.at[bi, :, pl.ds(lo, _CG)], xbuf)

            def group(g, _):
                lblv = lbuf[pl.ds(c * _CG + g * 16, 16)]
                xs = [xbuf[j, pl.ds(g * 16, 16)] for j in range(e)]
                plsc.addupdate_scatter(accs[e], [lblv], onesf)
                for j in range(e):
                    plsc.addupdate_scatter(accs[j], [lblv], xs[j])
                return 0

            lax.fori_loop(0, _CG // 16, group, 0)
            return 0

        lax.fori_loop(0, p // _CG, chunk, 0)
        for j in range(nb):
            pltpu.sync_copy(
                accs[j],
                out_ref.at[pl.ds(((bi * nw + wid) * nb + j) * _K, _K)])


def _tc_body(emb_ref, lab_ref, part_ref, lp_ref, lq_ref, lr_ref, kp_ref,
             sumsT_s, counts_s, cl_s, *, nt, nw):
    t = pl.program_id(1)
    tt = emb_ref.shape[2]
    e = emb_ref.shape[1]
    nb = e + 1
    cc = 512 if tt % 512 == 0 else tt
    nck = tt // cc
    kiota = lax.broadcasted_iota(jnp.int32, (_K, cc), 0)
    unit_cnt = (lax.broadcasted_iota(jnp.int32, (nb, 1), 0) == e
                ).astype(jnp.float32)

    @pl.when(t == 0)
    def _init():
        pr = part_ref[...]
        psum = jnp.sum(pr.reshape(nw, nb, _K), axis=0)
        sumsT_s[...] = psum[0:e, :]
        counts_s[...] = lax.dot_general(
            psum, unit_cnt, (((0,), (0,)), ((), ())),
            preferred_element_type=jnp.float32)
        cl_s[...] = jnp.zeros_like(cl_s)

    counts_col = jnp.maximum(counts_s[...], 1.0)
    counts_row = counts_col.reshape(1, _K)
    centersT = sumsT_s[...] / counts_row
    cn2_col = lax.dot_general(
        centersT * centersT, jnp.ones((e, 1), jnp.float32),
        (((0,), (0,)), ((), ())),
        preferred_element_type=jnp.float32)
    for c in range(nck):
        x = emb_ref[0, :, c * cc:(c + 1) * cc]
        lbl = lab_ref[0, 0, c * cc:(c + 1) * cc]
        oh = (lbl[None, :] == kiota).astype(jnp.float32)
        dots = lax.dot_general(
            centersT, x, (((0,), (0,)), ((), ())),
            preferred_element_type=jnp.float32)
        sel = jnp.sum(oh * (dots - 0.5 * cn2_col), axis=0)
        en2 = jnp.sum(x * x, axis=0)
        d2 = jnp.maximum(en2 - 2.0 * sel, 0.0) + _EPS
        dist = jnp.sqrt(d2)
        hinged = jnp.where(lbl > 0,
                           jnp.maximum(dist - _DELTA_PULL, 0.0) ** 2,
                           0.0)
        cl_s[...] += jnp.sum(oh * hinged[None, :], axis=1, keepdims=True)

    @pl.when(t == nt - 1)
    def _finalize():
        counts_cc = jnp.maximum(counts_s[...], 1.0)
        cenT = sumsT_s[...] / counts_cc.reshape(1, _K)
        kidx = lax.broadcasted_iota(jnp.int32, (_K, 1), 0)
        pf = jnp.where((counts_s[...] > 0.0) & (kidx >= 1), 1.0, 0.0)
        kp = jnp.sum(pf)
        kf = jnp.maximum(kp, 1.0)
        cen2 = lax.dot_general(
            cenT * cenT, jnp.ones((e, 1), jnp.float32),
            (((0,), (0,)), ((), ())),
            preferred_element_type=jnp.float32)
        l_pull = jnp.sum(pf * (cl_s[...] / counts_cc)) / kf
        norms = jnp.sqrt(cen2 + _EPS)
        l_reg = jnp.sum(pf * norms) / kf
        gram = lax.dot_general(
            cenT, cenT, (((0,), (0,)), ((), ())),
            preferred_element_type=jnp.float32)
        pw2 = jnp.maximum(cen2 + cen2.reshape(1, _K) - 2.0 * gram, 0.0)
        pw = jnp.sqrt(pw2 + _EPS)
        ii = lax.broadcasted_iota(jnp.int32, (_K, _K), 0)
        jj = lax.broadcasted_iota(jnp.int32, (_K, _K), 1)
        pair_f = lax.dot_general(
            pf, pf, (((1,), (1,)), ((), ())),
            preferred_element_type=jnp.float32)
        pair_f = pair_f * jnp.where(jj > ii, 1.0, 0.0)
        hv = pair_f * jnp.maximum(2.0 * _DELTA_PUSH - pw, 0.0) ** 2
        npairs = jnp.sum(pair_f)
        l_push = jnp.where(npairs > 0.0,
                           jnp.sum(hv) / jnp.maximum(npairs, 1.0),
                           0.0)
        lp_ref[...] = jnp.reshape(l_pull, (1, 1, 1))
        lq_ref[...] = jnp.reshape(l_push, (1, 1, 1))
        lr_ref[...] = jnp.reshape(l_reg, (1, 1, 1))
        kp_ref[...] = jnp.reshape(kp, (1, 1, 1))


def kernel(embedding, ins_label):
    b, e = embedding.shape[0], embedding.shape[1]
    n = embedding.shape[2] * embedding.shape[3]
    t = 8192 if n % 8192 == 0 else n
    nt = n // t
    nb = e + 1
    emb = embedding.reshape(b, e, n)
    lab2 = ins_label.reshape(b, n)
    lab = ins_label.reshape(b * nt, 1, t)

    info = plsc.get_sparse_core_info()
    nw = info.num_cores * info.num_subcores
    p = n // nw

    sc_seg = functools.partial(
        pl.kernel,
        mesh=plsc.VectorSubcoreMesh(core_axis_name="c", subcore_axis_name="s"),
        out_type=jax.ShapeDtypeStruct((b * nw * nb * _K,), jnp.float32),
        compiler_params=pltpu.CompilerParams(needs_layout_passes=False),
        scratch_types=[
            pltpu.VMEM((p,), jnp.int32),
            pltpu.VMEM((e, _CG), jnp.float32),
        ] + [pltpu.VMEM((_K,), jnp.float32) for _ in range(nb)],
    )(functools.partial(_sc_body, b=b, e=e, n=n, nw=nw))
    partials = sc_seg(emb, lab2).reshape(b * nw * nb, _K)

    out_shape = [jax.ShapeDtypeStruct((b, 1, 1), jnp.float32)] * 4
    out_spec = pl.BlockSpec((1, 1, 1), lambda bi, ti: (bi, 0, 0))
    lp, lq, lr, kp = pl.pallas_call(
        functools.partial(_tc_body, nt=nt, nw=nw),
        grid=(b, nt),
        in_specs=[
            pl.BlockSpec((1, e, t), lambda bi, ti: (bi, 0, ti)),
            pl.BlockSpec((1, 1, t), lambda bi, ti: (bi * nt + ti, 0, 0)),
            pl.BlockSpec((nw * nb, _K), lambda bi, ti: (bi, 0)),
        ],
        out_specs=[out_spec] * 4,
        out_shape=out_shape,
        scratch_shapes=[
            pltpu.VMEM((e, _K), jnp.float32),
            pltpu.VMEM((_K, 1), jnp.float32),
            pltpu.VMEM((_K, 1), jnp.float32),
        ],
    )(emb, lab, partials)

    lp = lp.reshape(b)
    lq = lq.reshape(b)
    lr = lr.reshape(b)
    kp = kp.reshape(b)
    has = (kp > 0.0).astype(jnp.float32)
    nvalid = jnp.maximum(jnp.sum(has), 1.0)
    l_pull = jnp.sum(has * lp) / nvalid
    l_push = jnp.sum(has * lq) / nvalid
    l_reg = jnp.sum(has * lr) / nvalid
    total = _W_PULL * l_pull + _W_PUSH * l_push + _W_REG * l_reg
    return {"loss": total, "l_pull": l_pull, "l_push": l_push,
            "l_reg": l_reg}
